# baseline (speedup 1.0000x reference)
import functools

import jax
import jax.numpy as jnp
from jax import lax
from jax.experimental import pallas as pl
from jax.experimental.pallas import tpu as pltpu
from jax.experimental.pallas import tpu_sc as plsc

_G = 16
_N = 256
_D = 128
_B = 4
_ITEMS = 256
_EPOCHS = 3
_LR = 0.01
_STEPS = _EPOCHS * _ITEMS


def _gram_body(emb_ref, nodes_ref, d0_ref, xx_ref, sqn0_ref):
    nodes = nodes_ref[...]
    sqn0_ref[...] = jnp.sum(nodes * nodes, axis=1, keepdims=True)
    for b in range(_B):
        xb = emb_ref[b]
        d0_ref[b, :, :] = jax.lax.dot_general(
            nodes, xb, (((1,), (1,)), ((), ())),
            preferred_element_type=jnp.float32,
            precision=jax.lax.Precision.HIGHEST)
        xx_ref[b, :, :] = jax.lax.dot_general(
            xb, xb, (((1,), (1,)), ((), ())),
            preferred_element_type=jnp.float32,
            precision=jax.lax.Precision.HIGHEST)


def _gram(embeddings, nodes_flat):
    return pl.pallas_call(
        _gram_body,
        out_shape=(
            jax.ShapeDtypeStruct((_B, _N, _ITEMS), jnp.float32),
            jax.ShapeDtypeStruct((_B, _ITEMS, _ITEMS), jnp.float32),
            jax.ShapeDtypeStruct((_N, 1), jnp.float32),
        ),
    )(embeddings, nodes_flat)


_mesh = plsc.VectorSubcoreMesh(core_axis_name="c", subcore_axis_name="s")


@functools.partial(
    pl.kernel,
    out_type=jax.ShapeDtypeStruct((_B, _D), jnp.float32),
    mesh=_mesh,
    scratch_types=[
        pltpu.VMEM((_N, _ITEMS), jnp.float32),
        pltpu.VMEM((_N * _D,), jnp.float32),
        pltpu.VMEM((16, 16), jnp.float32),
        pltpu.VMEM((2, _ITEMS), jnp.float32),
        pltpu.VMEM((2, _D), jnp.float32),
        pltpu.VMEM((_D,), jnp.float32),
        pltpu.SemaphoreType.DMA,
        pltpu.SemaphoreType.DMA,
        pltpu.SemaphoreType.DMA,
    ],
    compiler_params=pltpu.CompilerParams(needs_layout_passes=False),
)
def _som_sc(emb_hbm, nodes_hbm, sqn0_hbm, d0_hbm, xx_hbm, out_hbm,
            d_v, g_v, sqn_v, xx_v, x_v, out_v, sem_xx, sem_x, sem_p):
    wid = lax.axis_index("s") * 2 + lax.axis_index("c")

    @pl.when(wid < _B)
    def _body():
        b = wid
        iota = lax.iota(jnp.int32, 16)

        cp1 = pltpu.async_copy(d0_hbm.at[b], d_v, sem_p)
        cp2 = pltpu.async_copy(nodes_hbm, g_v, sem_p)
        cp3 = pltpu.async_copy(sqn0_hbm, sqn_v, sem_p)
        pltpu.async_copy(xx_hbm.at[b, 0], xx_v.at[0], sem_xx)
        pltpu.async_copy(emb_hbm.at[b, 0], x_v.at[0], sem_x)
        cp1.wait()
        cp2.wait()
        cp3.wait()

        one = jnp.int32(1)
        zero = jnp.int32(0)
        dxv = (jnp.where(iota == 1, one, zero)
               - jnp.where(iota == 3, one, zero))
        dyv = (jnp.where(iota == 0, one, zero)
               - jnp.where(iota == 2, one, zero))
        lane_lt4 = iota < 4

        def step(s, carry):
            t = s & (_ITEMS - 1)
            slot = s & 1
            nslot = 1 - slot
            tn = (s + 1) & (_ITEMS - 1)
            tvec = jnp.full((16,), t, jnp.int32)
            pltpu.make_async_copy(xx_hbm.at[b, t], xx_v.at[slot],
                                  sem_xx).wait()
            pltpu.make_async_copy(emb_hbm.at[b, t], x_v.at[slot],
                                  sem_x).wait()
            pltpu.async_copy(xx_hbm.at[b, tn], xx_v.at[nslot], sem_xx)
            pltpu.async_copy(emb_hbm.at[b, tn], x_v.at[nslot], sem_x)

            scores = []
            macc = None
            for c in range(16):
                dcol = plsc.load_gather(d_v, [iota + (c * 16), tvec])
                sc = sqn_v[c, :] - 2.0 * dcol
                scores.append(sc)
                macc = sc if macc is None else jnp.minimum(macc, sc)
            m = jnp.min(macc)
            cacc = None
            for c in range(16):
                cand = jnp.where(scores[c] == m, iota + (c * 16),
                                 jnp.int32(_N))
                cacc = cand if cacc is None else jnp.minimum(cacc, cand)
            bmu1 = jnp.min(cacc)

            m2acc = None
            for c in range(16):
                scm = jnp.where(iota + (c * 16) == bmu1,
                                jnp.float32(3.0e38), scores[c])
                m2acc = scm if m2acc is None else jnp.minimum(m2acc, scm)
            m2 = jnp.min(m2acc)
            cacc2 = None
            for c in range(16):
                scm = jnp.where(iota + (c * 16) == bmu1,
                                jnp.float32(3.0e38), scores[c])
                cand2 = jnp.where(scm == m2, iota + (c * 16), jnp.int32(_N))
                cacc2 = cand2 if cacc2 is None else jnp.minimum(cacc2, cand2)
            bmu2 = jnp.min(cacc2)

            def exact_d2(r):
                rb = r * _D
                acc = None
                for jc in range(8):
                    diff = (g_v[pl.ds(rb + jc * 16, 16)]
                            - x_v[slot, pl.ds(jc * 16, 16)])
                    sq = diff * diff
                    acc = sq if acc is None else acc + sq
                return jnp.sum(acc)

            d1 = exact_d2(bmu1)
            d2 = exact_d2(bmu2)
            take2 = (d2 < d1) | ((d2 == d1) & (bmu2 < bmu1))
            bmu = jnp.where(take2, bmu2, bmu1)

            bx = bmu >> 4
            by = bmu & 15
            nxv = bx + dxv
            nyv = by + dyv
            validv = (lane_lt4 & (nxv >= 0) & (nxv < _G)
                      & (nyv >= 0) & (nyv < _G))
            rv = (jnp.clip(nxv, 0, _G - 1) * _G
                  + jnp.clip(nyv, 0, _G - 1))
            cvec = jnp.where(validv, jnp.float32(_LR), jnp.float32(0.0))

            drow = plsc.load_gather(d_v, [rv, tvec])
            xx_tt = plsc.load_gather(
                xx_v, [jnp.full((16,), slot, jnp.int32), tvec])
            sqn_old = plsc.load_gather(sqn_v, [rv >> 4, rv & 15])
            omc = 1.0 - cvec
            sqn_new = (omc * omc * sqn_old + 2.0 * cvec * omc * drow
                       + cvec * cvec * xx_tt)
            plsc.store_scatter(sqn_v, [rv >> 4, rv & 15], sqn_new,
                               mask=validv)

            for j in range(4):
                r = lax.squeeze(lax.slice(rv, (j,), (j + 1,)), (0,))
                cr = lax.squeeze(lax.slice(cvec, (j,), (j + 1,)), (0,))
                rb_g = r * _D
                for cchunk in range(16):
                    xxc = xx_v[slot, pl.ds(cchunk * 16, 16)]
                    dsl = d_v[r, pl.ds(cchunk * 16, 16)]
                    d_v[r, pl.ds(cchunk * 16, 16)] = dsl + cr * (xxc - dsl)
                for jc in range(8):
                    xj = x_v[slot, pl.ds(jc * 16, 16)]
                    gs = g_v[pl.ds(rb_g + jc * 16, 16)]
                    g_v[pl.ds(rb_g + jc * 16, 16)] = gs + cr * (xj - gs)
            return carry

        lax.fori_loop(0, _STEPS, step, 0)
        pltpu.make_async_copy(xx_hbm.at[b, 0], xx_v.at[0], sem_xx).wait()
        pltpu.make_async_copy(emb_hbm.at[b, 0], x_v.at[0], sem_x).wait()

        def acc_row(r, accs):
            return tuple(accs[j] + g_v[pl.ds(r * _D + j * 16, 16)]
                         for j in range(8))
        accs = tuple(jnp.zeros((16,), jnp.float32) for _ in range(8))
        accs = lax.fori_loop(0, _N, acc_row, accs)
        for j in range(8):
            out_v[pl.ds(j * 16, 16)] = accs[j]
        pltpu.sync_copy(out_v, out_hbm.at[b])


def kernel(embeddings, nodes):
    nodes_flat = nodes.reshape(_N, _D)
    d0, xx, sqn0 = _gram(embeddings, nodes_flat)
    out = _som_sc(
        embeddings,
        nodes_flat.reshape(_N * _D),
        sqn0.reshape(16, 16),
        d0,
        xx,
    )
    return out


# speedup vs baseline: 1.0126x; 1.0126x over previous
import functools

import jax
import jax.numpy as jnp
from jax import lax
from jax.experimental import pallas as pl
from jax.experimental.pallas import tpu as pltpu
from jax.experimental.pallas import tpu_sc as plsc

_G = 16
_N = 256
_D = 128
_B = 4
_ITEMS = 256
_EPOCHS = 3
_LR = 0.01
_STEPS = _EPOCHS * _ITEMS


def _gram_body(emb_ref, nodes_ref, d0_ref, xx_ref, sqn0_ref):
    nodes = nodes_ref[...]
    sqn0_ref[...] = jnp.sum(nodes * nodes, axis=1, keepdims=True)
    for b in range(_B):
        xb = emb_ref[b]
        d0_ref[b, :, :] = jax.lax.dot_general(
            nodes, xb, (((1,), (1,)), ((), ())),
            preferred_element_type=jnp.float32,
            precision=jax.lax.Precision.HIGHEST)
        xx_ref[b, :, :] = jax.lax.dot_general(
            xb, xb, (((1,), (1,)), ((), ())),
            preferred_element_type=jnp.float32,
            precision=jax.lax.Precision.HIGHEST)


def _gram(embeddings, nodes_flat):
    return pl.pallas_call(
        _gram_body,
        out_shape=(
            jax.ShapeDtypeStruct((_B, _N, _ITEMS), jnp.float32),
            jax.ShapeDtypeStruct((_B, _ITEMS, _ITEMS), jnp.float32),
            jax.ShapeDtypeStruct((_N, 1), jnp.float32),
        ),
    )(embeddings, nodes_flat)


def _tree_min(vs):
    while len(vs) > 1:
        vs = [jnp.minimum(vs[i], vs[i + 1]) for i in range(0, len(vs), 2)]
    return vs[0]


_mesh = plsc.VectorSubcoreMesh(core_axis_name="c", subcore_axis_name="s")


@functools.partial(
    pl.kernel,
    out_type=jax.ShapeDtypeStruct((_B, _D), jnp.float32),
    mesh=_mesh,
    scratch_types=[
        pltpu.VMEM((_N, _ITEMS), jnp.float32),
        pltpu.VMEM((_N * _D,), jnp.float32),
        pltpu.VMEM((16, 16), jnp.float32),
        pltpu.VMEM((2, _ITEMS), jnp.float32),
        pltpu.VMEM((2, _D), jnp.float32),
        pltpu.VMEM((_D,), jnp.float32),
        pltpu.SemaphoreType.DMA,
        pltpu.SemaphoreType.DMA,
        pltpu.SemaphoreType.DMA,
    ],
    compiler_params=pltpu.CompilerParams(needs_layout_passes=False),
)
def _som_sc(emb_hbm, nodes_hbm, sqn0_hbm, d0_hbm, xx_hbm, out_hbm,
            d_v, g_v, sqn_v, xx_v, x_v, out_v, sem_xx, sem_x, sem_p):
    wid = lax.axis_index("s") * 2 + lax.axis_index("c")

    @pl.when(wid < _B)
    def _body():
        b = wid
        iota = lax.iota(jnp.int32, 16)

        cp1 = pltpu.async_copy(d0_hbm.at[b], d_v, sem_p)
        cp2 = pltpu.async_copy(nodes_hbm, g_v, sem_p)
        cp3 = pltpu.async_copy(sqn0_hbm, sqn_v, sem_p)
        pltpu.async_copy(xx_hbm.at[b, 0], xx_v.at[0], sem_xx)
        pltpu.async_copy(emb_hbm.at[b, 0], x_v.at[0], sem_x)
        cp1.wait()
        cp2.wait()
        cp3.wait()

        lane_lt4 = iota < 4

        def step(s, carry):
            t = s & (_ITEMS - 1)
            slot = s & 1
            nslot = 1 - slot
            tn = (s + 1) & (_ITEMS - 1)
            tvec = jnp.full((16,), t, jnp.int32)

            keys = []
            for c in range(16):
                dcol = plsc.load_gather(d_v, [iota + (c * 16), tvec])
                sc = sqn_v[c, :] - 2.0 * dcol
                bits = plsc.bitcast(sc, jnp.int32)
                sortable = bits ^ (lax.shift_right_arithmetic(bits, 31)
                                   & jnp.int32(0x7FFFFFFF))
                keys.append((sortable & jnp.int32(-256)) | (iota + (c * 16)))
            key1 = jnp.min(_tree_min(keys))
            bmu1 = key1 & 255
            masked = [jnp.where(k == key1, jnp.int32(0x7FFFFFFF), k)
                      for k in keys]
            key2 = jnp.min(_tree_min(masked))
            bmu2 = key2 & 255

            pltpu.make_async_copy(xx_hbm.at[b, t], xx_v.at[slot],
                                  sem_xx).wait()
            pltpu.make_async_copy(emb_hbm.at[b, t], x_v.at[slot],
                                  sem_x).wait()
            pltpu.async_copy(xx_hbm.at[b, tn], xx_v.at[nslot], sem_xx)
            pltpu.async_copy(emb_hbm.at[b, tn], x_v.at[nslot], sem_x)

            acc1 = None
            acc2 = None
            rb1 = bmu1 * _D
            rb2 = bmu2 * _D
            for jc in range(8):
                xc = x_v[slot, pl.ds(jc * 16, 16)]
                df1 = g_v[pl.ds(rb1 + jc * 16, 16)] - xc
                df2 = g_v[pl.ds(rb2 + jc * 16, 16)] - xc
                sq1 = df1 * df1
                sq2 = df2 * df2
                acc1 = sq1 if acc1 is None else acc1 + sq1
                acc2 = sq2 if acc2 is None else acc2 + sq2
            d1 = jnp.sum(acc1)
            d2 = jnp.sum(acc2)
            take2 = (d2 < d1) | ((d2 == d1) & (bmu2 < bmu1))
            bmu = jnp.where(take2, bmu2, bmu1)

            bx = bmu >> 4
            by = bmu & 15
            bxv = bx + (jnp.where(iota == 1, 1, 0)
                        - jnp.where(iota == 3, 1, 0))
            byv = by + (jnp.where(iota == 0, 1, 0)
                        - jnp.where(iota == 2, 1, 0))
            validv = (lane_lt4 & (bxv >= 0) & (bxv < _G)
                      & (byv >= 0) & (byv < _G))
            rv = (jnp.clip(bxv, 0, _G - 1) * _G
                  + jnp.clip(byv, 0, _G - 1))
            cvec = jnp.where(validv, jnp.float32(_LR), jnp.float32(0.0))

            drow = plsc.load_gather(d_v, [rv, tvec])
            xx_tt = plsc.load_gather(
                xx_v, [jnp.full((16,), slot, jnp.int32), tvec])
            sqn_old = plsc.load_gather(sqn_v, [rv >> 4, rv & 15])
            omc = 1.0 - cvec
            sqn_new = (omc * omc * sqn_old + 2.0 * cvec * omc * drow
                       + cvec * cvec * xx_tt)
            plsc.store_scatter(sqn_v, [rv >> 4, rv & 15], sqn_new,
                               mask=validv)

            rs = []
            crs = []
            for dx, dy in ((0, 1), (1, 0), (0, -1), (-1, 0)):
                nx = bx + dx
                ny = by + dy
                ok = ((nx >= 0) & (nx < _G) & (ny >= 0) & (ny < _G))
                r = (jnp.clip(nx, 0, _G - 1) * _G
                     + jnp.clip(ny, 0, _G - 1))
                rs.append(r)
                crs.append(jnp.where(ok, jnp.float32(_LR), jnp.float32(0.0)))

            for cchunk in range(16):
                xxc = xx_v[slot, pl.ds(cchunk * 16, 16)]
                for j in range(4):
                    dsl = d_v[rs[j], pl.ds(cchunk * 16, 16)]
                    d_v[rs[j], pl.ds(cchunk * 16, 16)] = (
                        dsl + crs[j] * (xxc - dsl))
            for jc in range(8):
                xc = x_v[slot, pl.ds(jc * 16, 16)]
                for j in range(4):
                    gb = rs[j] * _D + jc * 16
                    gs = g_v[pl.ds(gb, 16)]
                    g_v[pl.ds(gb, 16)] = gs + crs[j] * (xc - gs)
            return carry

        lax.fori_loop(0, _STEPS, step, 0)
        pltpu.make_async_copy(xx_hbm.at[b, 0], xx_v.at[0], sem_xx).wait()
        pltpu.make_async_copy(emb_hbm.at[b, 0], x_v.at[0], sem_x).wait()

        def acc_row(r, accs):
            return tuple(accs[j] + g_v[pl.ds(r * _D + j * 16, 16)]
                         for j in range(8))
        accs = tuple(jnp.zeros((16,), jnp.float32) for _ in range(8))
        accs = lax.fori_loop(0, _N, acc_row, accs)
        for j in range(8):
            out_v[pl.ds(j * 16, 16)] = accs[j]
        pltpu.sync_copy(out_v, out_hbm.at[b])


def kernel(embeddings, nodes):
    nodes_flat = nodes.reshape(_N, _D)
    d0, xx, sqn0 = _gram(embeddings, nodes_flat)
    out = _som_sc(
        embeddings,
        nodes_flat.reshape(_N * _D),
        sqn0.reshape(16, 16),
        d0,
        xx,
    )
    return out


# speedup vs baseline: 1.1257x; 1.1117x over previous
import functools

import jax
import jax.numpy as jnp
from jax import lax
from jax.experimental import pallas as pl
from jax.experimental.pallas import tpu as pltpu
from jax.experimental.pallas import tpu_sc as plsc

_G = 16
_N = 256
_D = 128
_B = 4
_ITEMS = 256
_EPOCHS = 3
_LR = 0.01
_STEPS = _EPOCHS * _ITEMS


def _gram_body(emb_ref, nodes_ref, d0_ref, xx_ref, sqn0_ref):
    nodes = nodes_ref[...]
    sqn0_ref[...] = jnp.sum(nodes * nodes, axis=1, keepdims=True)
    for b in range(_B):
        xb = emb_ref[b]
        d0_ref[b, :, 0:256] = jax.lax.dot_general(
            nodes, xb, (((1,), (1,)), ((), ())),
            preferred_element_type=jnp.float32,
            precision=jax.lax.Precision.HIGHEST)
        xx_ref[b, :, :] = jax.lax.dot_general(
            xb, xb, (((1,), (1,)), ((), ())),
            preferred_element_type=jnp.float32,
            precision=jax.lax.Precision.HIGHEST)


def _gram(embeddings, nodes_flat):
    return pl.pallas_call(
        _gram_body,
        out_shape=(
            jax.ShapeDtypeStruct((_B, _N, _ITEMS + 1), jnp.float32),
            jax.ShapeDtypeStruct((_B, _ITEMS, _ITEMS), jnp.float32),
            jax.ShapeDtypeStruct((_N, 1), jnp.float32),
        ),
    )(embeddings, nodes_flat)


def _tree_min(vs):
    while len(vs) > 1:
        vs = [jnp.minimum(vs[i], vs[i + 1]) for i in range(0, len(vs), 2)]
    return vs[0]


_mesh = plsc.VectorSubcoreMesh(core_axis_name="c", subcore_axis_name="s")


@functools.partial(
    pl.kernel,
    out_type=jax.ShapeDtypeStruct((_B, _D), jnp.float32),
    mesh=_mesh,
    scratch_types=[
        pltpu.VMEM((_N, _ITEMS + 1), jnp.float32),
        pltpu.VMEM((_N * _D,), jnp.float32),
        pltpu.VMEM((16, 16), jnp.float32),
        pltpu.VMEM((2, _ITEMS), jnp.float32),
        pltpu.VMEM((2, _D), jnp.float32),
        pltpu.VMEM((_D,), jnp.float32),
        pltpu.SemaphoreType.DMA,
        pltpu.SemaphoreType.DMA,
        pltpu.SemaphoreType.DMA,
    ],
    compiler_params=pltpu.CompilerParams(needs_layout_passes=False,
                                         use_tc_tiling_on_sc=False),
)
def _som_sc(emb_hbm, nodes_hbm, sqn0_hbm, d0_hbm, xx_hbm, out_hbm,
            d_v, g_v, sqn_v, xx_v, x_v, out_v, sem_xx, sem_x, sem_p):
    wid = lax.axis_index("s") * 2 + lax.axis_index("c")

    @pl.when(wid < _B)
    def _body():
        b = wid
        iota = lax.iota(jnp.int32, 16)

        cp1 = pltpu.async_copy(d0_hbm.at[b], d_v, sem_p)
        cp2 = pltpu.async_copy(nodes_hbm, g_v, sem_p)
        cp3 = pltpu.async_copy(sqn0_hbm, sqn_v, sem_p)
        pltpu.async_copy(xx_hbm.at[b, 0], xx_v.at[0], sem_xx)
        pltpu.async_copy(emb_hbm.at[b, 0], x_v.at[0], sem_x)
        cp1.wait()
        cp2.wait()
        cp3.wait()

        lane_lt4 = iota < 4

        def step(s, carry):
            t = s & (_ITEMS - 1)
            slot = s & 1
            nslot = 1 - slot
            tn = (s + 1) & (_ITEMS - 1)
            tvec = jnp.full((16,), t, jnp.int32)

            keys = []
            for c in range(16):
                dcol = plsc.load_gather(d_v, [iota + (c * 16), tvec])
                sc = sqn_v[c, :] - 2.0 * dcol
                bits = plsc.bitcast(sc, jnp.int32)
                sortable = bits ^ (lax.shift_right_arithmetic(bits, 31)
                                   & jnp.int32(0x7FFFFFFF))
                keys.append((sortable & jnp.int32(-256)) | (iota + (c * 16)))
            key1 = jnp.min(_tree_min(keys))
            bmu1 = key1 & 255
            masked = [jnp.where(k == key1, jnp.int32(0x7FFFFFFF), k)
                      for k in keys]
            key2 = jnp.min(_tree_min(masked))
            bmu2 = key2 & 255

            pltpu.make_async_copy(xx_hbm.at[b, t], xx_v.at[slot],
                                  sem_xx).wait()
            pltpu.make_async_copy(emb_hbm.at[b, t], x_v.at[slot],
                                  sem_x).wait()
            pltpu.async_copy(xx_hbm.at[b, tn], xx_v.at[nslot], sem_xx)
            pltpu.async_copy(emb_hbm.at[b, tn], x_v.at[nslot], sem_x)

            acc1 = None
            acc2 = None
            rb1 = bmu1 * _D
            rb2 = bmu2 * _D
            for jc in range(8):
                xc = x_v[slot, pl.ds(jc * 16, 16)]
                df1 = g_v[pl.ds(rb1 + jc * 16, 16)] - xc
                df2 = g_v[pl.ds(rb2 + jc * 16, 16)] - xc
                sq1 = df1 * df1
                sq2 = df2 * df2
                acc1 = sq1 if acc1 is None else acc1 + sq1
                acc2 = sq2 if acc2 is None else acc2 + sq2
            d1 = jnp.sum(acc1)
            d2 = jnp.sum(acc2)
            take2 = (d2 < d1) | ((d2 == d1) & (bmu2 < bmu1))
            bmu = jnp.where(take2, bmu2, bmu1)

            bx = bmu >> 4
            by = bmu & 15
            bxv = bx + (jnp.where(iota == 1, 1, 0)
                        - jnp.where(iota == 3, 1, 0))
            byv = by + (jnp.where(iota == 0, 1, 0)
                        - jnp.where(iota == 2, 1, 0))
            validv = (lane_lt4 & (bxv >= 0) & (bxv < _G)
                      & (byv >= 0) & (byv < _G))
            rv = (jnp.clip(bxv, 0, _G - 1) * _G
                  + jnp.clip(byv, 0, _G - 1))
            cvec = jnp.where(validv, jnp.float32(_LR), jnp.float32(0.0))

            drow = plsc.load_gather(d_v, [rv, tvec])
            xx_tt = plsc.load_gather(
                xx_v, [jnp.full((16,), slot, jnp.int32), tvec])
            sqn_old = plsc.load_gather(sqn_v, [rv >> 4, rv & 15])
            omc = 1.0 - cvec
            sqn_new = (omc * omc * sqn_old + 2.0 * cvec * omc * drow
                       + cvec * cvec * xx_tt)
            plsc.store_scatter(sqn_v, [rv >> 4, rv & 15], sqn_new,
                               mask=validv)

            rs = []
            crs = []
            for dx, dy in ((0, 1), (1, 0), (0, -1), (-1, 0)):
                nx = bx + dx
                ny = by + dy
                ok = ((nx >= 0) & (nx < _G) & (ny >= 0) & (ny < _G))
                r = (jnp.clip(nx, 0, _G - 1) * _G
                     + jnp.clip(ny, 0, _G - 1))
                rs.append(r)
                crs.append(jnp.where(ok, jnp.float32(_LR), jnp.float32(0.0)))

            for cchunk in range(16):
                xxc = xx_v[slot, pl.ds(cchunk * 16, 16)]
                for j in range(4):
                    dsl = d_v[rs[j], pl.ds(cchunk * 16, 16)]
                    d_v[rs[j], pl.ds(cchunk * 16, 16)] = (
                        dsl + crs[j] * (xxc - dsl))
            for jc in range(8):
                xc = x_v[slot, pl.ds(jc * 16, 16)]
                for j in range(4):
                    gb = rs[j] * _D + jc * 16
                    gs = g_v[pl.ds(gb, 16)]
                    g_v[pl.ds(gb, 16)] = gs + crs[j] * (xc - gs)
            return carry

        lax.fori_loop(0, _STEPS, step, 0)
        pltpu.make_async_copy(xx_hbm.at[b, 0], xx_v.at[0], sem_xx).wait()
        pltpu.make_async_copy(emb_hbm.at[b, 0], x_v.at[0], sem_x).wait()

        def acc_row(r, accs):
            return tuple(accs[j] + g_v[pl.ds(r * _D + j * 16, 16)]
                         for j in range(8))
        accs = tuple(jnp.zeros((16,), jnp.float32) for _ in range(8))
        accs = lax.fori_loop(0, _N, acc_row, accs)
        for j in range(8):
            out_v[pl.ds(j * 16, 16)] = accs[j]
        pltpu.sync_copy(out_v, out_hbm.at[b])


def kernel(embeddings, nodes):
    nodes_flat = nodes.reshape(_N, _D)
    d0, xx, sqn0 = _gram(embeddings, nodes_flat)
    out = _som_sc(
        embeddings,
        nodes_flat.reshape(_N * _D),
        sqn0.reshape(16, 16),
        d0,
        xx,
    )
    return out


# speedup vs baseline: 1.7979x; 1.5971x over previous
import functools

import jax
import jax.numpy as jnp
from jax import lax
from jax.experimental import pallas as pl
from jax.experimental.pallas import tpu as pltpu
from jax.experimental.pallas import tpu_sc as plsc

_G = 16
_N = 256
_D = 128
_B = 4
_ITEMS = 256
_EPOCHS = 3
_LR = 0.01
_STEPS = _EPOCHS * _ITEMS


def _gram_body(emb_ref, nodes_ref, d0_ref, xx_ref, sqn0_ref):
    nodes = nodes_ref[...]
    sqn0_ref[...] = jnp.sum(nodes * nodes, axis=1, keepdims=True)
    for b in range(_B):
        xb = emb_ref[b]
        d0_ref[b, :, 0:256] = jax.lax.dot_general(
            nodes, xb, (((1,), (1,)), ((), ())),
            preferred_element_type=jnp.float32,
            precision=jax.lax.Precision.HIGHEST)
        xx_ref[b, :, :] = jax.lax.dot_general(
            xb, xb, (((1,), (1,)), ((), ())),
            preferred_element_type=jnp.float32,
            precision=jax.lax.Precision.HIGHEST)


def _gram(embeddings, nodes_flat):
    return pl.pallas_call(
        _gram_body,
        out_shape=(
            jax.ShapeDtypeStruct((_B, _N, _ITEMS + 1), jnp.float32),
            jax.ShapeDtypeStruct((_B, _ITEMS, _ITEMS), jnp.float32),
            jax.ShapeDtypeStruct((_N, 1), jnp.float32),
        ),
    )(embeddings, nodes_flat)


def _tree_min(vs):
    while len(vs) > 1:
        vs = [jnp.minimum(vs[i], vs[i + 1]) for i in range(0, len(vs), 2)]
    return vs[0]


_mesh = plsc.VectorSubcoreMesh(core_axis_name="c", subcore_axis_name="s")


@functools.partial(
    pl.kernel,
    out_type=jax.ShapeDtypeStruct((_B, _D), jnp.float32),
    mesh=_mesh,
    scratch_types=[
        pltpu.VMEM((_N, _ITEMS + 1), jnp.float32),
        pltpu.VMEM((_N * _D,), jnp.float32),
        pltpu.VMEM((16, 16), jnp.float32),
        pltpu.VMEM((2, _ITEMS), jnp.float32),
        pltpu.VMEM((2, _D), jnp.float32),
        pltpu.VMEM((_D,), jnp.float32),
        pltpu.SemaphoreType.DMA,
        pltpu.SemaphoreType.DMA,
        pltpu.SemaphoreType.DMA,
    ],
    compiler_params=pltpu.CompilerParams(needs_layout_passes=False,
                                         use_tc_tiling_on_sc=False),
)
def _som_sc(emb_hbm, nodes_hbm, sqn0_hbm, d0_hbm, xx_hbm, out_hbm,
            d_v, g_v, sqn_v, xx_v, x_v, out_v, sem_xx, sem_x, sem_p):
    wid = lax.axis_index("s") * 2 + lax.axis_index("c")

    @pl.when(wid < _B)
    def _body():
        b = wid
        iota = lax.iota(jnp.int32, 16)

        cp1 = pltpu.async_copy(d0_hbm.at[b], d_v, sem_p)
        cp2 = pltpu.async_copy(nodes_hbm, g_v, sem_p)
        cp3 = pltpu.async_copy(sqn0_hbm, sqn_v, sem_p)
        pltpu.async_copy(xx_hbm.at[b, 0], xx_v.at[0], sem_xx)
        pltpu.async_copy(emb_hbm.at[b, 0], x_v.at[0], sem_x)
        cp1.wait()
        cp2.wait()
        cp3.wait()

        lane_lt4 = iota < 4

        def step(s, carry):
            t = s & (_ITEMS - 1)
            slot = s & 1
            nslot = 1 - slot
            tn = (s + 1) & (_ITEMS - 1)
            tvec = jnp.full((16,), t, jnp.int32)

            keys = []
            for c in range(16):
                dcol = plsc.load_gather(d_v, [iota + (c * 16), tvec])
                sc = sqn_v[c, :] - 2.0 * dcol
                bits = plsc.bitcast(sc, jnp.int32)
                sortable = bits ^ (lax.shift_right_arithmetic(bits, 31)
                                   & jnp.int32(0x7FFFFFFF))
                keys.append((sortable & jnp.int32(-256)) | (iota + (c * 16)))
            key1 = jnp.min(_tree_min(keys))
            bmu1 = key1 & 255
            masked = [jnp.where(k == key1, jnp.int32(0x7FFFFFFF), k)
                      for k in keys]
            key2 = jnp.min(_tree_min(masked))
            bmu2 = key2 & 255

            pltpu.make_async_copy(xx_hbm.at[b, t], xx_v.at[slot],
                                  sem_xx).wait()
            pltpu.make_async_copy(emb_hbm.at[b, t], x_v.at[slot],
                                  sem_x).wait()
            pltpu.async_copy(xx_hbm.at[b, tn], xx_v.at[nslot], sem_xx)
            pltpu.async_copy(emb_hbm.at[b, tn], x_v.at[nslot], sem_x)

            acc1 = None
            acc2 = None
            rb1 = bmu1 * _D
            rb2 = bmu2 * _D
            for jc in range(8):
                xc = x_v[slot, pl.ds(jc * 16, 16)]
                df1 = g_v[pl.ds(rb1 + jc * 16, 16)] - xc
                df2 = g_v[pl.ds(rb2 + jc * 16, 16)] - xc
                sq1 = df1 * df1
                sq2 = df2 * df2
                acc1 = sq1 if acc1 is None else acc1 + sq1
                acc2 = sq2 if acc2 is None else acc2 + sq2
            d1 = jnp.sum(acc1)
            d2 = jnp.sum(acc2)
            take2 = (d2 < d1) | ((d2 == d1) & (bmu2 < bmu1))
            bmu = jnp.where(take2, bmu2, bmu1)

            bx = bmu >> 4
            by = bmu & 15
            bxv = bx + (jnp.where(iota == 1, 1, 0)
                        - jnp.where(iota == 3, 1, 0))
            byv = by + (jnp.where(iota == 0, 1, 0)
                        - jnp.where(iota == 2, 1, 0))
            validv = (lane_lt4 & (bxv >= 0) & (bxv < _G)
                      & (byv >= 0) & (byv < _G))
            rv = (jnp.clip(bxv, 0, _G - 1) * _G
                  + jnp.clip(byv, 0, _G - 1))
            cvec = jnp.where(validv, jnp.float32(_LR), jnp.float32(0.0))

            drow = plsc.load_gather(d_v, [rv, tvec])
            xx_tt = plsc.load_gather(
                xx_v, [jnp.full((16,), slot, jnp.int32), tvec])
            sqn_old = plsc.load_gather(sqn_v, [rv >> 4, rv & 15])
            omc = 1.0 - cvec
            sqn_new = (omc * omc * sqn_old + 2.0 * cvec * omc * drow
                       + cvec * cvec * xx_tt)
            plsc.store_scatter(sqn_v, [rv >> 4, rv & 15], sqn_new,
                               mask=validv)

            rs = []
            crs = []
            for dx, dy in ((0, 1), (1, 0), (0, -1), (-1, 0)):
                nx = bx + dx
                ny = by + dy
                ok = ((nx >= 0) & (nx < _G) & (ny >= 0) & (ny < _G))
                r = (jnp.clip(nx, 0, _G - 1) * _G
                     + jnp.clip(ny, 0, _G - 1))
                rs.append(r)
                crs.append(jnp.where(ok, jnp.float32(_LR), jnp.float32(0.0)))

            for cchunk in range(16):
                xxc = xx_v[slot, pl.ds(cchunk * 16, 16)]
                dsls = [d_v[rs[j], pl.ds(cchunk * 16, 16)]
                        for j in range(4)]
                news = [dsls[j] + crs[j] * (xxc - dsls[j])
                        for j in range(4)]
                for j in range(4):
                    d_v[rs[j], pl.ds(cchunk * 16, 16)] = news[j]
            for jc in range(8):
                xc = x_v[slot, pl.ds(jc * 16, 16)]
                gss = [g_v[pl.ds(rs[j] * _D + jc * 16, 16)]
                       for j in range(4)]
                ngs = [gss[j] + crs[j] * (xc - gss[j]) for j in range(4)]
                for j in range(4):
                    g_v[pl.ds(rs[j] * _D + jc * 16, 16)] = ngs[j]
            return carry

        lax.fori_loop(0, _STEPS, step, 0)
        pltpu.make_async_copy(xx_hbm.at[b, 0], xx_v.at[0], sem_xx).wait()
        pltpu.make_async_copy(emb_hbm.at[b, 0], x_v.at[0], sem_x).wait()

        def acc_row(r, accs):
            return tuple(accs[j] + g_v[pl.ds(r * _D + j * 16, 16)]
                         for j in range(8))
        accs = tuple(jnp.zeros((16,), jnp.float32) for _ in range(8))
        accs = lax.fori_loop(0, _N, acc_row, accs)
        for j in range(8):
            out_v[pl.ds(j * 16, 16)] = accs[j]
        pltpu.sync_copy(out_v, out_hbm.at[b])


def kernel(embeddings, nodes):
    nodes_flat = nodes.reshape(_N, _D)
    d0, xx, sqn0 = _gram(embeddings, nodes_flat)
    out = _som_sc(
        embeddings,
        nodes_flat.reshape(_N * _D),
        sqn0.reshape(16, 16),
        d0,
        xx,
    )
    return out


# speedup vs baseline: 2.2850x; 1.2709x over previous
import functools

import jax
import jax.numpy as jnp
from jax import lax
from jax.experimental import pallas as pl
from jax.experimental.pallas import tpu as pltpu
from jax.experimental.pallas import tpu_sc as plsc

_G = 16
_N = 256
_D = 128
_B = 4
_ITEMS = 256
_EPOCHS = 3
_LR = 0.01
_STEPS = _EPOCHS * _ITEMS


def _gram_body(emb_ref, nodes_ref, d0_ref, xx_ref, sqn0_ref):
    nodes = nodes_ref[...]
    sqn0_ref[...] = jnp.sum(nodes * nodes, axis=1, keepdims=True)
    for b in range(_B):
        xb = emb_ref[b]
        d0_ref[b, :, 0:256] = jax.lax.dot_general(
            nodes, xb, (((1,), (1,)), ((), ())),
            preferred_element_type=jnp.float32,
            precision=jax.lax.Precision.HIGHEST)
        xx_ref[b, :, :] = jax.lax.dot_general(
            xb, xb, (((1,), (1,)), ((), ())),
            preferred_element_type=jnp.float32,
            precision=jax.lax.Precision.HIGHEST)


def _gram(embeddings, nodes_flat):
    return pl.pallas_call(
        _gram_body,
        out_shape=(
            jax.ShapeDtypeStruct((_B, _N, _ITEMS + 1), jnp.float32),
            jax.ShapeDtypeStruct((_B, _ITEMS, _ITEMS), jnp.float32),
            jax.ShapeDtypeStruct((_N, 1), jnp.float32),
        ),
    )(embeddings, nodes_flat)


def _tree_min(vs):
    while len(vs) > 1:
        vs = [jnp.minimum(vs[i], vs[i + 1]) for i in range(0, len(vs), 2)]
    return vs[0]


_mesh = plsc.VectorSubcoreMesh(core_axis_name="c", subcore_axis_name="s")


@functools.partial(
    pl.kernel,
    out_type=jax.ShapeDtypeStruct((_B, _D), jnp.float32),
    mesh=_mesh,
    scratch_types=[
        pltpu.VMEM((_N, _ITEMS + 1), jnp.float32),
        pltpu.VMEM((_N * _D,), jnp.float32),
        pltpu.VMEM((16, 16), jnp.float32),
        pltpu.VMEM((2, _ITEMS), jnp.float32),
        pltpu.VMEM((2, _D), jnp.float32),
        pltpu.VMEM((_D,), jnp.float32),
        pltpu.VMEM_SHARED((2, _ITEMS, _ITEMS), jnp.float32),
        pltpu.VMEM_SHARED((2, _ITEMS, _D), jnp.float32),
        pltpu.SemaphoreType.DMA,
        pltpu.SemaphoreType.DMA,
        pltpu.SemaphoreType.DMA,
    ],
    compiler_params=pltpu.CompilerParams(needs_layout_passes=False,
                                         use_tc_tiling_on_sc=False),
)
def _som_sc(emb_hbm, nodes_hbm, sqn0_hbm, d0_hbm, xx_hbm, out_hbm,
            d_v, g_v, sqn_v, xx_v, x_v, out_v, xx_sp, x_sp,
            sem_xx, sem_x, sem_p):
    wid = lax.axis_index("s") * 2 + lax.axis_index("c")

    @pl.when(wid < _B)
    def _body():
        b = wid
        iota = lax.iota(jnp.int32, 16)

        lidx = wid >> 1
        cp1 = pltpu.async_copy(d0_hbm.at[b], d_v, sem_p)
        cp2 = pltpu.async_copy(nodes_hbm, g_v, sem_p)
        cp3 = pltpu.async_copy(sqn0_hbm, sqn_v, sem_p)
        cp4 = pltpu.async_copy(xx_hbm.at[b], xx_sp.at[lidx], sem_p)
        cp5 = pltpu.async_copy(emb_hbm.at[b], x_sp.at[lidx], sem_p)
        cp1.wait()
        cp2.wait()
        cp3.wait()
        cp4.wait()
        cp5.wait()
        pltpu.async_copy(xx_sp.at[lidx, 0], xx_v.at[0], sem_xx)
        pltpu.async_copy(x_sp.at[lidx, 0], x_v.at[0], sem_x)

        lane_lt4 = iota < 4

        def step(s, carry):
            t = s & (_ITEMS - 1)
            slot = s & 1
            nslot = 1 - slot
            tn = (s + 1) & (_ITEMS - 1)
            tvec = jnp.full((16,), t, jnp.int32)

            keys = []
            for c in range(16):
                dcol = plsc.load_gather(d_v, [iota + (c * 16), tvec])
                sc = sqn_v[c, :] - 2.0 * dcol
                bits = plsc.bitcast(sc, jnp.int32)
                sortable = bits ^ (lax.shift_right_arithmetic(bits, 31)
                                   & jnp.int32(0x7FFFFFFF))
                keys.append((sortable & jnp.int32(-256)) | (iota + (c * 16)))
            key1 = jnp.min(_tree_min(keys))
            bmu1 = key1 & 255
            masked = [jnp.where(k == key1, jnp.int32(0x7FFFFFFF), k)
                      for k in keys]
            key2 = jnp.min(_tree_min(masked))
            bmu2 = key2 & 255

            pltpu.make_async_copy(xx_sp.at[lidx, t], xx_v.at[slot],
                                  sem_xx).wait()
            pltpu.make_async_copy(x_sp.at[lidx, t], x_v.at[slot],
                                  sem_x).wait()
            pltpu.async_copy(xx_sp.at[lidx, tn], xx_v.at[nslot], sem_xx)
            pltpu.async_copy(x_sp.at[lidx, tn], x_v.at[nslot], sem_x)

            acc1 = None
            acc2 = None
            rb1 = bmu1 * _D
            rb2 = bmu2 * _D
            for jc in range(8):
                xc = x_v[slot, pl.ds(jc * 16, 16)]
                df1 = g_v[pl.ds(rb1 + jc * 16, 16)] - xc
                df2 = g_v[pl.ds(rb2 + jc * 16, 16)] - xc
                sq1 = df1 * df1
                sq2 = df2 * df2
                acc1 = sq1 if acc1 is None else acc1 + sq1
                acc2 = sq2 if acc2 is None else acc2 + sq2
            d1 = jnp.sum(acc1)
            d2 = jnp.sum(acc2)
            take2 = (d2 < d1) | ((d2 == d1) & (bmu2 < bmu1))
            bmu = jnp.where(take2, bmu2, bmu1)

            bx = bmu >> 4
            by = bmu & 15
            bxv = bx + (jnp.where(iota == 1, 1, 0)
                        - jnp.where(iota == 3, 1, 0))
            byv = by + (jnp.where(iota == 0, 1, 0)
                        - jnp.where(iota == 2, 1, 0))
            validv = (lane_lt4 & (bxv >= 0) & (bxv < _G)
                      & (byv >= 0) & (byv < _G))
            rv = (jnp.clip(bxv, 0, _G - 1) * _G
                  + jnp.clip(byv, 0, _G - 1))
            cvec = jnp.where(validv, jnp.float32(_LR), jnp.float32(0.0))

            drow = plsc.load_gather(d_v, [rv, tvec])
            xx_tt = plsc.load_gather(
                xx_v, [jnp.full((16,), slot, jnp.int32), tvec])
            sqn_old = plsc.load_gather(sqn_v, [rv >> 4, rv & 15])
            omc = 1.0 - cvec
            sqn_new = (omc * omc * sqn_old + 2.0 * cvec * omc * drow
                       + cvec * cvec * xx_tt)
            plsc.store_scatter(sqn_v, [rv >> 4, rv & 15], sqn_new,
                               mask=validv)

            rs = []
            crs = []
            for dx, dy in ((0, 1), (1, 0), (0, -1), (-1, 0)):
                nx = bx + dx
                ny = by + dy
                ok = ((nx >= 0) & (nx < _G) & (ny >= 0) & (ny < _G))
                r = (jnp.clip(nx, 0, _G - 1) * _G
                     + jnp.clip(ny, 0, _G - 1))
                rs.append(r)
                crs.append(jnp.where(ok, jnp.float32(_LR), jnp.float32(0.0)))

            for cchunk in range(16):
                xxc = xx_v[slot, pl.ds(cchunk * 16, 16)]
                dsls = [d_v[rs[j], pl.ds(cchunk * 16, 16)]
                        for j in range(4)]
                news = [dsls[j] + crs[j] * (xxc - dsls[j])
                        for j in range(4)]
                for j in range(4):
                    d_v[rs[j], pl.ds(cchunk * 16, 16)] = news[j]
            for jc in range(8):
                xc = x_v[slot, pl.ds(jc * 16, 16)]
                gss = [g_v[pl.ds(rs[j] * _D + jc * 16, 16)]
                       for j in range(4)]
                ngs = [gss[j] + crs[j] * (xc - gss[j]) for j in range(4)]
                for j in range(4):
                    g_v[pl.ds(rs[j] * _D + jc * 16, 16)] = ngs[j]
            return carry

        lax.fori_loop(0, _STEPS, step, 0)
        pltpu.make_async_copy(xx_sp.at[lidx, 0], xx_v.at[0], sem_xx).wait()
        pltpu.make_async_copy(x_sp.at[lidx, 0], x_v.at[0], sem_x).wait()

        def acc_row(r, accs):
            return tuple(accs[j] + g_v[pl.ds(r * _D + j * 16, 16)]
                         for j in range(8))
        accs = tuple(jnp.zeros((16,), jnp.float32) for _ in range(8))
        accs = lax.fori_loop(0, _N, acc_row, accs)
        for j in range(8):
            out_v[pl.ds(j * 16, 16)] = accs[j]
        pltpu.sync_copy(out_v, out_hbm.at[b])


def kernel(embeddings, nodes):
    nodes_flat = nodes.reshape(_N, _D)
    d0, xx, sqn0 = _gram(embeddings, nodes_flat)
    out = _som_sc(
        embeddings,
        nodes_flat.reshape(_N * _D),
        sqn0.reshape(16, 16),
        d0,
        xx,
    )
    return out


# speedup vs baseline: 2.5372x; 1.1104x over previous
import functools

import jax
import jax.numpy as jnp
from jax import lax
from jax.experimental import pallas as pl
from jax.experimental.pallas import tpu as pltpu
from jax.experimental.pallas import tpu_sc as plsc

_G = 16
_N = 256
_D = 128
_B = 4
_ITEMS = 256
_EPOCHS = 3
_LR = 0.01
_STEPS = _EPOCHS * _ITEMS


def _gram_body(emb_ref, nodes_ref, d0_ref, xx_ref, sqn0_ref):
    nodes = nodes_ref[...]
    sqn0_ref[...] = jnp.sum(nodes * nodes, axis=1, keepdims=True)
    for b in range(_B):
        xb = emb_ref[b]
        d0_ref[b, :, 0:256] = jax.lax.dot_general(
            nodes, xb, (((1,), (1,)), ((), ())),
            preferred_element_type=jnp.float32,
            precision=jax.lax.Precision.HIGHEST)
        xx_ref[b, :, :] = jax.lax.dot_general(
            xb, xb, (((1,), (1,)), ((), ())),
            preferred_element_type=jnp.float32,
            precision=jax.lax.Precision.HIGHEST)


def _gram(embeddings, nodes_flat):
    return pl.pallas_call(
        _gram_body,
        out_shape=(
            jax.ShapeDtypeStruct((_B, _N, _ITEMS + 1), jnp.float32),
            jax.ShapeDtypeStruct((_B, _ITEMS, _ITEMS), jnp.float32),
            jax.ShapeDtypeStruct((_N, 1), jnp.float32),
        ),
    )(embeddings, nodes_flat)


def _tree_min(vs):
    while len(vs) > 1:
        vs = [jnp.minimum(vs[i], vs[i + 1]) for i in range(0, len(vs), 2)]
    return vs[0]


_mesh = plsc.VectorSubcoreMesh(core_axis_name="c", subcore_axis_name="s")


@functools.partial(
    pl.kernel,
    out_type=jax.ShapeDtypeStruct((_B, _D), jnp.float32),
    mesh=_mesh,
    scratch_types=[
        pltpu.VMEM((_N, _ITEMS + 1), jnp.float32),
        pltpu.VMEM((_N * _D,), jnp.float32),
        pltpu.VMEM((16, 16), jnp.float32),
        pltpu.VMEM((16, 16), jnp.float32),
        pltpu.VMEM((2, _ITEMS), jnp.float32),
        pltpu.VMEM((2, _D), jnp.float32),
        pltpu.VMEM((_D,), jnp.float32),
        pltpu.VMEM_SHARED((2, _ITEMS, _ITEMS), jnp.float32),
        pltpu.VMEM_SHARED((2, _ITEMS, _D), jnp.float32),
        pltpu.SemaphoreType.DMA,
        pltpu.SemaphoreType.DMA,
        pltpu.SemaphoreType.DMA,
    ],
    compiler_params=pltpu.CompilerParams(needs_layout_passes=False,
                                         use_tc_tiling_on_sc=False),
)
def _som_sc(emb_hbm, nodes_hbm, sqn0_hbm, d0_hbm, xx_hbm, out_hbm,
            d_v, g_v, sqn_v, lam_v, xx_v, x_v, out_v, xx_sp, x_sp,
            sem_xx, sem_x, sem_p):
    wid = lax.axis_index("s") * 2 + lax.axis_index("c")

    @pl.when(wid < _B)
    def _body():
        b = wid
        iota = lax.iota(jnp.int32, 16)

        lidx = wid >> 1
        cp1 = pltpu.async_copy(d0_hbm.at[b], d_v, sem_p)
        cp2 = pltpu.async_copy(nodes_hbm, g_v, sem_p)
        cp3 = pltpu.async_copy(sqn0_hbm, sqn_v, sem_p)
        cp4 = pltpu.async_copy(xx_hbm.at[b], xx_sp.at[lidx], sem_p)
        cp5 = pltpu.async_copy(emb_hbm.at[b], x_sp.at[lidx], sem_p)
        cp1.wait()
        cp2.wait()
        cp3.wait()
        cp4.wait()
        cp5.wait()
        pltpu.async_copy(xx_sp.at[lidx, 0], xx_v.at[0], sem_xx)
        pltpu.async_copy(x_sp.at[lidx, 0], x_v.at[0], sem_x)

        lane_lt4 = iota < 4
        ones16 = jnp.full((16,), 1.0, jnp.float32)
        for c in range(16):
            lam_v[c, :] = ones16

        def step(s, carry):
            t = s & (_ITEMS - 1)
            slot = s & 1
            nslot = 1 - slot
            tn = (s + 1) & (_ITEMS - 1)
            tvec = jnp.full((16,), t, jnp.int32)

            keys = []
            for c in range(16):
                dcol = plsc.load_gather(d_v, [iota + (c * 16), tvec])
                sc = sqn_v[c, :] - 2.0 * (lam_v[c, :] * dcol)
                bits = plsc.bitcast(sc, jnp.int32)
                sortable = bits ^ (lax.shift_right_arithmetic(bits, 31)
                                   & jnp.int32(0x7FFFFFFF))
                keys.append((sortable & jnp.int32(-256)) | (iota + (c * 16)))
            key1 = jnp.min(_tree_min(keys))
            bmu1 = key1 & 255
            masked = [jnp.where(k == key1, jnp.int32(0x7FFFFFFF), k)
                      for k in keys]
            key2 = jnp.min(_tree_min(masked))
            bmu2 = key2 & 255

            pltpu.make_async_copy(xx_sp.at[lidx, t], xx_v.at[slot],
                                  sem_xx).wait()
            pltpu.make_async_copy(x_sp.at[lidx, t], x_v.at[slot],
                                  sem_x).wait()
            pltpu.async_copy(xx_sp.at[lidx, tn], xx_v.at[nslot], sem_xx)
            pltpu.async_copy(x_sp.at[lidx, tn], x_v.at[nslot], sem_x)

            acc1 = None
            acc2 = None
            rb1 = bmu1 * _D
            rb2 = bmu2 * _D
            for jc in range(8):
                xc = x_v[slot, pl.ds(jc * 16, 16)]
                df1 = g_v[pl.ds(rb1 + jc * 16, 16)] - xc
                df2 = g_v[pl.ds(rb2 + jc * 16, 16)] - xc
                sq1 = df1 * df1
                sq2 = df2 * df2
                acc1 = sq1 if acc1 is None else acc1 + sq1
                acc2 = sq2 if acc2 is None else acc2 + sq2
            d1 = jnp.sum(acc1)
            d2 = jnp.sum(acc2)
            take2 = (d2 < d1) | ((d2 == d1) & (bmu2 < bmu1))
            bmu = jnp.where(take2, bmu2, bmu1)

            bx = bmu >> 4
            by = bmu & 15
            bxv = bx + (jnp.where(iota == 1, 1, 0)
                        - jnp.where(iota == 3, 1, 0))
            byv = by + (jnp.where(iota == 0, 1, 0)
                        - jnp.where(iota == 2, 1, 0))
            validv = (lane_lt4 & (bxv >= 0) & (bxv < _G)
                      & (byv >= 0) & (byv < _G))
            rv = (jnp.clip(bxv, 0, _G - 1) * _G
                  + jnp.clip(byv, 0, _G - 1))
            cvec = jnp.where(validv, jnp.float32(_LR), jnp.float32(0.0))

            lam_old = plsc.load_gather(lam_v, [rv >> 4, rv & 15])
            drow = lam_old * plsc.load_gather(d_v, [rv, tvec])
            xx_tt = plsc.load_gather(
                xx_v, [jnp.full((16,), slot, jnp.int32), tvec])
            sqn_old = plsc.load_gather(sqn_v, [rv >> 4, rv & 15])
            omc = 1.0 - cvec
            sqn_new = (omc * omc * sqn_old + 2.0 * cvec * omc * drow
                       + cvec * cvec * xx_tt)
            plsc.store_scatter(sqn_v, [rv >> 4, rv & 15], sqn_new,
                               mask=validv)
            lam_new = omc * lam_old
            plsc.store_scatter(lam_v, [rv >> 4, rv & 15], lam_new,
                               mask=validv)
            addv = cvec / lam_new

            rs = []
            crs = []
            adds = []
            for j, (dx, dy) in enumerate(((0, 1), (1, 0), (0, -1), (-1, 0))):
                nx = bx + dx
                ny = by + dy
                ok = ((nx >= 0) & (nx < _G) & (ny >= 0) & (ny < _G))
                r = (jnp.clip(nx, 0, _G - 1) * _G
                     + jnp.clip(ny, 0, _G - 1))
                rs.append(r)
                crs.append(jnp.where(ok, jnp.float32(_LR), jnp.float32(0.0)))
                adds.append(lax.squeeze(lax.slice(addv, (j,), (j + 1,)),
                                        (0,)))

            for cchunk in range(16):
                xxc = xx_v[slot, pl.ds(cchunk * 16, 16)]
                for j in range(4):
                    plsc.addupdate(d_v.at[rs[j], pl.ds(cchunk * 16, 16)],
                                   adds[j] * xxc)
            for jc in range(8):
                xc = x_v[slot, pl.ds(jc * 16, 16)]
                gss = [g_v[pl.ds(rs[j] * _D + jc * 16, 16)]
                       for j in range(4)]
                ngs = [gss[j] + crs[j] * (xc - gss[j]) for j in range(4)]
                for j in range(4):
                    g_v[pl.ds(rs[j] * _D + jc * 16, 16)] = ngs[j]
            return carry

        lax.fori_loop(0, _STEPS, step, 0)
        pltpu.make_async_copy(xx_sp.at[lidx, 0], xx_v.at[0], sem_xx).wait()
        pltpu.make_async_copy(x_sp.at[lidx, 0], x_v.at[0], sem_x).wait()

        def acc_row(r, accs):
            return tuple(accs[j] + g_v[pl.ds(r * _D + j * 16, 16)]
                         for j in range(8))
        accs = tuple(jnp.zeros((16,), jnp.float32) for _ in range(8))
        accs = lax.fori_loop(0, _N, acc_row, accs)
        for j in range(8):
            out_v[pl.ds(j * 16, 16)] = accs[j]
        pltpu.sync_copy(out_v, out_hbm.at[b])


def kernel(embeddings, nodes):
    nodes_flat = nodes.reshape(_N, _D)
    d0, xx, sqn0 = _gram(embeddings, nodes_flat)
    out = _som_sc(
        embeddings,
        nodes_flat.reshape(_N * _D),
        sqn0.reshape(16, 16),
        d0,
        xx,
    )
    return out


# speedup vs baseline: 2.9714x; 1.1711x over previous
import functools

import jax
import jax.numpy as jnp
from jax import lax
from jax.experimental import pallas as pl
from jax.experimental.pallas import tpu as pltpu
from jax.experimental.pallas import tpu_sc as plsc

_G = 16
_N = 256
_D = 128
_B = 4
_ITEMS = 256
_EPOCHS = 3
_LR = 0.01
_STEPS = _EPOCHS * _ITEMS


def _gram_body(emb_ref, nodes_ref, d0_ref, xx_ref, sqn0_ref):
    nodes = nodes_ref[...]
    sqn0_ref[...] = jnp.sum(nodes * nodes, axis=1, keepdims=True)
    for b in range(_B):
        xb = emb_ref[b]
        d0_ref[b, :, 0:256] = jax.lax.dot_general(
            nodes, xb, (((1,), (1,)), ((), ())),
            preferred_element_type=jnp.float32,
            precision=jax.lax.Precision.HIGHEST)
        xx_ref[b, :, :] = jax.lax.dot_general(
            xb, xb, (((1,), (1,)), ((), ())),
            preferred_element_type=jnp.float32,
            precision=jax.lax.Precision.HIGHEST)


def _gram(embeddings, nodes_flat):
    return pl.pallas_call(
        _gram_body,
        out_shape=(
            jax.ShapeDtypeStruct((_B, _N, _ITEMS + 1), jnp.float32),
            jax.ShapeDtypeStruct((_B, _ITEMS, _ITEMS), jnp.float32),
            jax.ShapeDtypeStruct((_N, 1), jnp.float32),
        ),
    )(embeddings, nodes_flat)


def _tree_min(vs):
    while len(vs) > 1:
        vs = [jnp.minimum(vs[i], vs[i + 1]) for i in range(0, len(vs), 2)]
    return vs[0]


_mesh = plsc.VectorSubcoreMesh(core_axis_name="c", subcore_axis_name="s")


@functools.partial(
    pl.kernel,
    out_type=jax.ShapeDtypeStruct((_B, _D), jnp.float32),
    mesh=_mesh,
    scratch_types=[
        pltpu.VMEM((_N, _ITEMS + 1), jnp.float32),
        pltpu.VMEM((_N * _D,), jnp.float32),
        pltpu.VMEM((16, 16), jnp.float32),
        pltpu.VMEM((16, 16), jnp.float32),
        pltpu.VMEM((2, _ITEMS), jnp.float32),
        pltpu.VMEM((2, _D), jnp.float32),
        pltpu.VMEM((_D,), jnp.float32),
        pltpu.VMEM_SHARED((2, _ITEMS, _ITEMS), jnp.float32),
        pltpu.VMEM_SHARED((2, _ITEMS, _D), jnp.float32),
        pltpu.SemaphoreType.DMA,
        pltpu.SemaphoreType.DMA,
        pltpu.SemaphoreType.DMA,
    ],
    compiler_params=pltpu.CompilerParams(needs_layout_passes=False,
                                         use_tc_tiling_on_sc=False),
)
def _som_sc(emb_hbm, nodes_hbm, sqn0_hbm, d0_hbm, xx_hbm, out_hbm,
            d_v, g_v, sqn_v, lam_v, xx_v, x_v, out_v, xx_sp, x_sp,
            sem_xx, sem_x, sem_p):
    wid = lax.axis_index("s") * 2 + lax.axis_index("c")

    @pl.when(wid < _B)
    def _body():
        b = wid
        iota = lax.iota(jnp.int32, 16)

        lidx = wid >> 1
        cp1 = pltpu.async_copy(d0_hbm.at[b], d_v, sem_p)
        cp2 = pltpu.async_copy(nodes_hbm, g_v, sem_p)
        cp3 = pltpu.async_copy(sqn0_hbm, sqn_v, sem_p)
        cp4 = pltpu.async_copy(xx_hbm.at[b], xx_sp.at[lidx], sem_p)
        cp5 = pltpu.async_copy(emb_hbm.at[b], x_sp.at[lidx], sem_p)
        cp1.wait()
        cp2.wait()
        cp3.wait()
        cp4.wait()
        cp5.wait()
        pltpu.async_copy(xx_sp.at[lidx, 0], xx_v.at[0], sem_xx)
        pltpu.async_copy(x_sp.at[lidx, 0], x_v.at[0], sem_x)

        lane_lt4 = iota < 4
        ones16 = jnp.full((16,), 1.0, jnp.float32)
        for c in range(16):
            lam_v[c, :] = ones16

        def step(s, carry):
            t = s & (_ITEMS - 1)
            slot = s & 1
            nslot = 1 - slot
            tn = (s + 1) & (_ITEMS - 1)
            tvec = jnp.full((16,), t, jnp.int32)

            keys = []
            for c in range(16):
                dcol = plsc.load_gather(d_v, [iota + (c * 16), tvec])
                sc = sqn_v[c, :] - 2.0 * (lam_v[c, :] * dcol)
                bits = plsc.bitcast(sc, jnp.int32)
                sortable = bits ^ (lax.shift_right_arithmetic(bits, 31)
                                   & jnp.int32(0x7FFFFFFF))
                keys.append((sortable & jnp.int32(-256)) | (iota + (c * 16)))
            key1 = jnp.min(_tree_min(keys))
            bmu1 = key1 & 255
            masked = [jnp.where(k == key1, jnp.int32(0x7FFFFFFF), k)
                      for k in keys]
            key2 = jnp.min(_tree_min(masked))
            bmu2 = key2 & 255

            pltpu.make_async_copy(xx_sp.at[lidx, t], xx_v.at[slot],
                                  sem_xx).wait()
            pltpu.make_async_copy(x_sp.at[lidx, t], x_v.at[slot],
                                  sem_x).wait()
            pltpu.async_copy(xx_sp.at[lidx, tn], xx_v.at[nslot], sem_xx)
            pltpu.async_copy(x_sp.at[lidx, tn], x_v.at[nslot], sem_x)

            xcs = [x_v[slot, pl.ds(jc * 16, 16)] for jc in range(8)]
            acc1 = None
            acc2 = None
            rb1 = bmu1 * _D
            rb2 = bmu2 * _D
            for jc in range(8):
                xc = xcs[jc]
                df1 = g_v[pl.ds(rb1 + jc * 16, 16)] - xc
                df2 = g_v[pl.ds(rb2 + jc * 16, 16)] - xc
                sq1 = df1 * df1
                sq2 = df2 * df2
                acc1 = sq1 if acc1 is None else acc1 + sq1
                acc2 = sq2 if acc2 is None else acc2 + sq2
            d1 = jnp.sum(acc1)
            d2 = jnp.sum(acc2)
            take2 = (d2 < d1) | ((d2 == d1) & (bmu2 < bmu1))
            bmu = jnp.where(take2, bmu2, bmu1)

            bx = bmu >> 4
            by = bmu & 15
            bxv = bx + (jnp.where(iota == 1, 1, 0)
                        - jnp.where(iota == 3, 1, 0))
            byv = by + (jnp.where(iota == 0, 1, 0)
                        - jnp.where(iota == 2, 1, 0))
            validv = (lane_lt4 & (bxv >= 0) & (bxv < _G)
                      & (byv >= 0) & (byv < _G))
            rv = (jnp.clip(bxv, 0, _G - 1) * _G
                  + jnp.clip(byv, 0, _G - 1))
            cvec = jnp.where(validv, jnp.float32(_LR), jnp.float32(0.0))

            lam_old = plsc.load_gather(lam_v, [rv >> 4, rv & 15])
            drow = lam_old * plsc.load_gather(d_v, [rv, tvec])
            xx_tt = plsc.load_gather(
                xx_v, [jnp.full((16,), slot, jnp.int32), tvec])
            sqn_old = plsc.load_gather(sqn_v, [rv >> 4, rv & 15])
            omc = 1.0 - cvec
            sqn_new = (omc * omc * sqn_old + 2.0 * cvec * omc * drow
                       + cvec * cvec * xx_tt)
            plsc.store_scatter(sqn_v, [rv >> 4, rv & 15], sqn_new,
                               mask=validv)
            lam_new = omc * lam_old
            plsc.store_scatter(lam_v, [rv >> 4, rv & 15], lam_new,
                               mask=validv)
            addv = cvec / lam_new

            rs = []
            crs = []
            adds = []
            for j, (dx, dy) in enumerate(((0, 1), (1, 0), (0, -1), (-1, 0))):
                nx = bx + dx
                ny = by + dy
                ok = ((nx >= 0) & (nx < _G) & (ny >= 0) & (ny < _G))
                r = (jnp.clip(nx, 0, _G - 1) * _G
                     + jnp.clip(ny, 0, _G - 1))
                rs.append(r)
                crs.append(jnp.where(ok, jnp.float32(_LR), jnp.float32(0.0)))
                adds.append(lax.squeeze(lax.slice(addv, (j,), (j + 1,)),
                                        (0,)))

            xxcs = [xx_v[slot, pl.ds(cchunk * 16, 16)]
                    for cchunk in range(16)]
            for cchunk in range(16):
                for j in range(4):
                    plsc.addupdate(d_v.at[rs[j], pl.ds(cchunk * 16, 16)],
                                   adds[j] * xxcs[cchunk])
            for jc in range(8):
                xc = xcs[jc]
                gss = [g_v[pl.ds(rs[j] * _D + jc * 16, 16)]
                       for j in range(4)]
                ngs = [gss[j] + crs[j] * (xc - gss[j]) for j in range(4)]
                for j in range(4):
                    g_v[pl.ds(rs[j] * _D + jc * 16, 16)] = ngs[j]
            return carry

        lax.fori_loop(0, _STEPS, step, 0)
        pltpu.make_async_copy(xx_sp.at[lidx, 0], xx_v.at[0], sem_xx).wait()
        pltpu.make_async_copy(x_sp.at[lidx, 0], x_v.at[0], sem_x).wait()

        def acc_row(r, accs):
            return tuple(accs[j] + g_v[pl.ds(r * _D + j * 16, 16)]
                         for j in range(8))
        accs = tuple(jnp.zeros((16,), jnp.float32) for _ in range(8))
        accs = lax.fori_loop(0, _N, acc_row, accs)
        for j in range(8):
            out_v[pl.ds(j * 16, 16)] = accs[j]
        pltpu.sync_copy(out_v, out_hbm.at[b])


def kernel(embeddings, nodes):
    nodes_flat = nodes.reshape(_N, _D)
    d0, xx, sqn0 = _gram(embeddings, nodes_flat)
    out = _som_sc(
        embeddings,
        nodes_flat.reshape(_N * _D),
        sqn0.reshape(16, 16),
        d0,
        xx,
    )
    return out


# speedup vs baseline: 3.2667x; 1.0994x over previous
import functools

import jax
import jax.numpy as jnp
from jax import lax
from jax.experimental import pallas as pl
from jax.experimental.pallas import tpu as pltpu
from jax.experimental.pallas import tpu_sc as plsc

_G = 16
_N = 256
_D = 128
_B = 4
_ITEMS = 256
_EPOCHS = 3
_LR = 0.01
_STEPS = _EPOCHS * _ITEMS


def _gram_body(emb_ref, nodes_ref, d0_ref, xx_ref, sqn0_ref):
    nodes = nodes_ref[...]
    sqn0_ref[...] = jnp.sum(nodes * nodes, axis=1, keepdims=True)
    for b in range(_B):
        xb = emb_ref[b]
        d0_ref[b, :, 0:256] = jax.lax.dot_general(
            nodes, xb, (((1,), (1,)), ((), ())),
            preferred_element_type=jnp.float32,
            precision=jax.lax.Precision.HIGHEST)
        xx_ref[b, :, :] = jax.lax.dot_general(
            xb, xb, (((1,), (1,)), ((), ())),
            preferred_element_type=jnp.float32,
            precision=jax.lax.Precision.HIGHEST)


def _gram(embeddings, nodes_flat):
    return pl.pallas_call(
        _gram_body,
        out_shape=(
            jax.ShapeDtypeStruct((_B, _N, _ITEMS + 1), jnp.float32),
            jax.ShapeDtypeStruct((_B, _ITEMS, _ITEMS), jnp.float32),
            jax.ShapeDtypeStruct((_N, 1), jnp.float32),
        ),
    )(embeddings, nodes_flat)


def _tree_min(vs):
    while len(vs) > 1:
        vs = [jnp.minimum(vs[i], vs[i + 1]) for i in range(0, len(vs), 2)]
    return vs[0]


_mesh = plsc.VectorSubcoreMesh(core_axis_name="c", subcore_axis_name="s")


@functools.partial(
    pl.kernel,
    out_type=jax.ShapeDtypeStruct((_B, _D), jnp.float32),
    mesh=_mesh,
    scratch_types=[
        pltpu.VMEM((_N * (_ITEMS + 1),), jnp.float32),
        pltpu.VMEM((_N * _D,), jnp.float32),
        pltpu.VMEM((16, 16), jnp.float32),
        pltpu.VMEM((16, 16), jnp.float32),
        pltpu.VMEM((2, _ITEMS), jnp.float32),
        pltpu.VMEM((2, _D), jnp.float32),
        pltpu.VMEM((_D,), jnp.float32),
        pltpu.VMEM_SHARED((2, _ITEMS, _ITEMS), jnp.float32),
        pltpu.VMEM_SHARED((2, _ITEMS, _D), jnp.float32),
        pltpu.SemaphoreType.DMA,
        pltpu.SemaphoreType.DMA,
        pltpu.SemaphoreType.DMA,
    ],
    compiler_params=pltpu.CompilerParams(needs_layout_passes=False,
                                         use_tc_tiling_on_sc=False),
)
def _som_sc(emb_hbm, nodes_hbm, sqn0_hbm, d0_hbm, xx_hbm, out_hbm,
            d_v, g_v, sqn_v, lam_v, xx_v, x_v, out_v, xx_sp, x_sp,
            sem_xx, sem_x, sem_p):
    wid = lax.axis_index("s") * 2 + lax.axis_index("c")

    @pl.when(wid < _B)
    def _body():
        b = wid
        iota = lax.iota(jnp.int32, 16)

        lidx = wid >> 1
        cp1 = pltpu.async_copy(d0_hbm.at[b], d_v, sem_p)
        cp2 = pltpu.async_copy(nodes_hbm, g_v, sem_p)
        cp3 = pltpu.async_copy(sqn0_hbm, sqn_v, sem_p)
        cp4 = pltpu.async_copy(xx_hbm.at[b], xx_sp.at[lidx], sem_p)
        cp5 = pltpu.async_copy(emb_hbm.at[b], x_sp.at[lidx], sem_p)
        cp1.wait()
        cp2.wait()
        cp3.wait()
        cp4.wait()
        cp5.wait()
        pltpu.async_copy(xx_sp.at[lidx, 0], xx_v.at[0], sem_xx)
        pltpu.async_copy(x_sp.at[lidx, 0], x_v.at[0], sem_x)

        lane_lt4 = iota < 4
        ones16 = jnp.full((16,), 1.0, jnp.float32)
        for c in range(16):
            lam_v[c, :] = ones16

        def step(s, carry):
            t = s & (_ITEMS - 1)
            slot = s & 1
            nslot = 1 - slot
            tn = (s + 1) & (_ITEMS - 1)
            tvec = jnp.full((16,), t, jnp.int32)

            keys = []
            iota257 = iota * 257
            for c in range(16):
                dcol = plsc.load_gather(d_v, [iota257 + (c * 16 * 257 + t)])
                sc = sqn_v[c, :] - 2.0 * (lam_v[c, :] * dcol)
                bits = plsc.bitcast(sc, jnp.int32)
                sortable = bits ^ (lax.shift_right_arithmetic(bits, 31)
                                   & jnp.int32(0x7FFFFFFF))
                keys.append((sortable & jnp.int32(-256)) | (iota + (c * 16)))
            key1 = jnp.min(_tree_min(keys))
            bmu1 = key1 & 255
            masked = [jnp.where(k == key1, jnp.int32(0x7FFFFFFF), k)
                      for k in keys]
            key2 = jnp.min(_tree_min(masked))
            bmu2 = key2 & 255

            pltpu.make_async_copy(xx_sp.at[lidx, t], xx_v.at[slot],
                                  sem_xx).wait()
            pltpu.make_async_copy(x_sp.at[lidx, t], x_v.at[slot],
                                  sem_x).wait()
            pltpu.async_copy(xx_sp.at[lidx, tn], xx_v.at[nslot], sem_xx)
            pltpu.async_copy(x_sp.at[lidx, tn], x_v.at[nslot], sem_x)

            xcs = [x_v[slot, pl.ds(jc * 16, 16)] for jc in range(8)]
            acc1 = None
            acc2 = None
            rb1 = bmu1 * _D
            rb2 = bmu2 * _D
            for jc in range(8):
                xc = xcs[jc]
                df1 = g_v[pl.ds(rb1 + jc * 16, 16)] - xc
                df2 = g_v[pl.ds(rb2 + jc * 16, 16)] - xc
                sq1 = df1 * df1
                sq2 = df2 * df2
                acc1 = sq1 if acc1 is None else acc1 + sq1
                acc2 = sq2 if acc2 is None else acc2 + sq2
            d1 = jnp.sum(acc1)
            d2 = jnp.sum(acc2)
            take2 = (d2 < d1) | ((d2 == d1) & (bmu2 < bmu1))
            bmu = jnp.where(take2, bmu2, bmu1)

            bx = bmu >> 4
            by = bmu & 15
            bxv = bx + (jnp.where(iota == 1, 1, 0)
                        - jnp.where(iota == 3, 1, 0))
            byv = by + (jnp.where(iota == 0, 1, 0)
                        - jnp.where(iota == 2, 1, 0))
            validv = (lane_lt4 & (bxv >= 0) & (bxv < _G)
                      & (byv >= 0) & (byv < _G))
            rv = (jnp.clip(bxv, 0, _G - 1) * _G
                  + jnp.clip(byv, 0, _G - 1))
            cvec = jnp.where(validv, jnp.float32(_LR), jnp.float32(0.0))

            lam_old = plsc.load_gather(lam_v, [rv >> 4, rv & 15])
            drow = lam_old * plsc.load_gather(d_v, [rv * 257 + t])
            xx_tt = plsc.load_gather(
                xx_v, [jnp.full((16,), slot, jnp.int32), tvec])
            sqn_old = plsc.load_gather(sqn_v, [rv >> 4, rv & 15])
            omc = 1.0 - cvec
            sqn_new = (omc * omc * sqn_old + 2.0 * cvec * omc * drow
                       + cvec * cvec * xx_tt)
            plsc.store_scatter(sqn_v, [rv >> 4, rv & 15], sqn_new,
                               mask=validv)
            lam_new = omc * lam_old
            plsc.store_scatter(lam_v, [rv >> 4, rv & 15], lam_new,
                               mask=validv)
            addv = cvec / lam_new

            rs = []
            crs = []
            adds = []
            for j, (dx, dy) in enumerate(((0, 1), (1, 0), (0, -1), (-1, 0))):
                nx = bx + dx
                ny = by + dy
                ok = ((nx >= 0) & (nx < _G) & (ny >= 0) & (ny < _G))
                r = (jnp.clip(nx, 0, _G - 1) * _G
                     + jnp.clip(ny, 0, _G - 1))
                rs.append(r)
                crs.append(jnp.where(ok, jnp.float32(_LR), jnp.float32(0.0)))
                adds.append(lax.squeeze(lax.slice(addv, (j,), (j + 1,)),
                                        (0,)))

            xxcs = [xx_v[slot, pl.ds(cchunk * 16, 16)]
                    for cchunk in range(16)]
            for cchunk in range(16):
                for j in range(4):
                    plsc.addupdate(d_v.at[pl.ds(rs[j] * 257 + cchunk * 16, 16)],
                                   adds[j] * xxcs[cchunk])
            for jc in range(8):
                xc = xcs[jc]
                gss = [g_v[pl.ds(rs[j] * _D + jc * 16, 16)]
                       for j in range(4)]
                ngs = [gss[j] + crs[j] * (xc - gss[j]) for j in range(4)]
                for j in range(4):
                    g_v[pl.ds(rs[j] * _D + jc * 16, 16)] = ngs[j]
            return carry

        lax.fori_loop(0, _STEPS, step, 0)
        pltpu.make_async_copy(xx_sp.at[lidx, 0], xx_v.at[0], sem_xx).wait()
        pltpu.make_async_copy(x_sp.at[lidx, 0], x_v.at[0], sem_x).wait()

        def acc_row(r, accs):
            return tuple(accs[j] + g_v[pl.ds(r * _D + j * 16, 16)]
                         for j in range(8))
        accs = tuple(jnp.zeros((16,), jnp.float32) for _ in range(8))
        accs = lax.fori_loop(0, _N, acc_row, accs)
        for j in range(8):
            out_v[pl.ds(j * 16, 16)] = accs[j]
        pltpu.sync_copy(out_v, out_hbm.at[b])


def kernel(embeddings, nodes):
    nodes_flat = nodes.reshape(_N, _D)
    d0, xx, sqn0 = _gram(embeddings, nodes_flat)
    out = _som_sc(
        embeddings,
        nodes_flat.reshape(_N * _D),
        sqn0.reshape(16, 16),
        d0.reshape(_B, _N * (_ITEMS + 1)),
        xx,
    )
    return out


# speedup vs baseline: 3.2736x; 1.0021x over previous
import functools

import jax
import jax.numpy as jnp
from jax import lax
from jax.experimental import pallas as pl
from jax.experimental.pallas import tpu as pltpu
from jax.experimental.pallas import tpu_sc as plsc

_G = 16
_N = 256
_D = 128
_B = 4
_ITEMS = 256
_EPOCHS = 3
_LR = 0.01
_STEPS = _EPOCHS * _ITEMS


def _gram_body(emb_ref, nodes_ref, d0_ref, xx_ref, sqn0_ref):
    nodes = nodes_ref[...]
    sqn0_ref[...] = jnp.sum(nodes * nodes, axis=1, keepdims=True)
    for b in range(_B):
        xb = emb_ref[b]
        d0_ref[b, :, 0:256] = jax.lax.dot_general(
            nodes, xb, (((1,), (1,)), ((), ())),
            preferred_element_type=jnp.float32,
            precision=jax.lax.Precision.HIGHEST)
        xx_ref[b, :, :] = jax.lax.dot_general(
            xb, xb, (((1,), (1,)), ((), ())),
            preferred_element_type=jnp.float32,
            precision=jax.lax.Precision.HIGHEST)


def _gram(embeddings, nodes_flat):
    return pl.pallas_call(
        _gram_body,
        out_shape=(
            jax.ShapeDtypeStruct((_B, _N, _ITEMS + 1), jnp.float32),
            jax.ShapeDtypeStruct((_B, _ITEMS, _ITEMS), jnp.float32),
            jax.ShapeDtypeStruct((_N, 1), jnp.float32),
        ),
    )(embeddings, nodes_flat)


def _tree_min(vs):
    while len(vs) > 1:
        vs = [jnp.minimum(vs[i], vs[i + 1]) for i in range(0, len(vs), 2)]
    return vs[0]


_mesh = plsc.VectorSubcoreMesh(core_axis_name="c", subcore_axis_name="s")


@functools.partial(
    pl.kernel,
    out_type=jax.ShapeDtypeStruct((_B, _D), jnp.float32),
    mesh=_mesh,
    scratch_types=[
        pltpu.VMEM((_N * (_ITEMS + 1),), jnp.float32),
        pltpu.VMEM((_N * _D,), jnp.float32),
        pltpu.VMEM((_N,), jnp.float32),
        pltpu.VMEM((_N,), jnp.float32),
        pltpu.VMEM((2 * _ITEMS,), jnp.float32),
        pltpu.VMEM((2 * _D,), jnp.float32),
        pltpu.VMEM((_D,), jnp.float32),
        pltpu.VMEM_SHARED((2, _ITEMS, _ITEMS), jnp.float32),
        pltpu.VMEM_SHARED((2, _ITEMS, _D), jnp.float32),
        pltpu.SemaphoreType.DMA,
        pltpu.SemaphoreType.DMA,
        pltpu.SemaphoreType.DMA,
    ],
    compiler_params=pltpu.CompilerParams(needs_layout_passes=False,
                                         use_tc_tiling_on_sc=False),
)
def _som_sc(emb_hbm, nodes_hbm, sqn0_hbm, d0_hbm, xx_hbm, out_hbm,
            d_v, g_v, sqn_v, lam_v, xx_v, x_v, out_v, xx_sp, x_sp,
            sem_xx, sem_x, sem_p):
    wid = lax.axis_index("s") * 2 + lax.axis_index("c")

    @pl.when(wid < _B)
    def _body():
        b = wid
        iota = lax.iota(jnp.int32, 16)

        lidx = wid >> 1
        cp1 = pltpu.async_copy(d0_hbm.at[b], d_v, sem_p)
        cp2 = pltpu.async_copy(nodes_hbm, g_v, sem_p)
        cp3 = pltpu.async_copy(sqn0_hbm, sqn_v, sem_p)
        cp4 = pltpu.async_copy(xx_hbm.at[b], xx_sp.at[lidx], sem_p)
        cp5 = pltpu.async_copy(emb_hbm.at[b], x_sp.at[lidx], sem_p)
        cp1.wait()
        cp2.wait()
        cp3.wait()
        cp4.wait()
        cp5.wait()
        pltpu.async_copy(xx_sp.at[lidx, 0], xx_v.at[pl.ds(0, _ITEMS)], sem_xx)
        pltpu.async_copy(x_sp.at[lidx, 0], x_v.at[pl.ds(0, _D)], sem_x)

        lane_lt4 = iota < 4
        ones16 = jnp.full((16,), 1.0, jnp.float32)
        for c in range(16):
            lam_v[pl.ds(c * 16, 16)] = ones16

        def step(s, carry):
            t = s & (_ITEMS - 1)
            slot = s & 1
            nslot = 1 - slot
            tn = (s + 1) & (_ITEMS - 1)
            tvec = jnp.full((16,), t, jnp.int32)

            keys = []
            iota257 = iota * 257
            for c in range(16):
                dcol = plsc.load_gather(d_v, [iota257 + (c * 16 * 257 + t)])
                sc = (sqn_v[pl.ds(c * 16, 16)]
                      - 2.0 * (lam_v[pl.ds(c * 16, 16)] * dcol))
                bits = plsc.bitcast(sc, jnp.int32)
                sortable = bits ^ (lax.shift_right_arithmetic(bits, 31)
                                   & jnp.int32(0x7FFFFFFF))
                keys.append((sortable & jnp.int32(-256)) | (iota + (c * 16)))
            key1 = jnp.min(_tree_min(keys))
            bmu1 = key1 & 255
            masked = [jnp.where(k == key1, jnp.int32(0x7FFFFFFF), k)
                      for k in keys]
            key2 = jnp.min(_tree_min(masked))
            bmu2 = key2 & 255

            pltpu.make_async_copy(xx_sp.at[lidx, t],
                                  xx_v.at[pl.ds(slot * _ITEMS, _ITEMS)],
                                  sem_xx).wait()
            pltpu.make_async_copy(x_sp.at[lidx, t],
                                  x_v.at[pl.ds(slot * _D, _D)],
                                  sem_x).wait()
            pltpu.async_copy(xx_sp.at[lidx, tn],
                             xx_v.at[pl.ds(nslot * _ITEMS, _ITEMS)], sem_xx)
            pltpu.async_copy(x_sp.at[lidx, tn],
                             x_v.at[pl.ds(nslot * _D, _D)], sem_x)

            xcs = [x_v[pl.ds(slot * _D + jc * 16, 16)] for jc in range(8)]
            acc1 = None
            acc2 = None
            rb1 = bmu1 * _D
            rb2 = bmu2 * _D
            for jc in range(8):
                xc = xcs[jc]
                df1 = g_v[pl.ds(rb1 + jc * 16, 16)] - xc
                df2 = g_v[pl.ds(rb2 + jc * 16, 16)] - xc
                sq1 = df1 * df1
                sq2 = df2 * df2
                acc1 = sq1 if acc1 is None else acc1 + sq1
                acc2 = sq2 if acc2 is None else acc2 + sq2
            d1 = jnp.sum(acc1)
            d2 = jnp.sum(acc2)
            take2 = (d2 < d1) | ((d2 == d1) & (bmu2 < bmu1))
            bmu = jnp.where(take2, bmu2, bmu1)

            bx = bmu >> 4
            by = bmu & 15
            bxv = bx + (jnp.where(iota == 1, 1, 0)
                        - jnp.where(iota == 3, 1, 0))
            byv = by + (jnp.where(iota == 0, 1, 0)
                        - jnp.where(iota == 2, 1, 0))
            validv = (lane_lt4 & (bxv >= 0) & (bxv < _G)
                      & (byv >= 0) & (byv < _G))
            rv = (jnp.clip(bxv, 0, _G - 1) * _G
                  + jnp.clip(byv, 0, _G - 1))
            cvec = jnp.where(validv, jnp.float32(_LR), jnp.float32(0.0))

            lam_old = plsc.load_gather(lam_v, [rv])
            drow = lam_old * plsc.load_gather(d_v, [rv * 257 + t])
            xx_tt = plsc.load_gather(
                xx_v, [jnp.full((16,), slot * _ITEMS + t, jnp.int32)])
            sqn_old = plsc.load_gather(sqn_v, [rv])
            omc = 1.0 - cvec
            sqn_new = (omc * omc * sqn_old + 2.0 * cvec * omc * drow
                       + cvec * cvec * xx_tt)
            plsc.store_scatter(sqn_v, [rv], sqn_new, mask=validv)
            lam_new = omc * lam_old
            plsc.store_scatter(lam_v, [rv], lam_new, mask=validv)
            addv = cvec / lam_new

            rs = []
            crs = []
            adds = []
            for j, (dx, dy) in enumerate(((0, 1), (1, 0), (0, -1), (-1, 0))):
                nx = bx + dx
                ny = by + dy
                ok = ((nx >= 0) & (nx < _G) & (ny >= 0) & (ny < _G))
                r = (jnp.clip(nx, 0, _G - 1) * _G
                     + jnp.clip(ny, 0, _G - 1))
                rs.append(r)
                crs.append(jnp.where(ok, jnp.float32(_LR), jnp.float32(0.0)))
                adds.append(lax.squeeze(lax.slice(addv, (j,), (j + 1,)),
                                        (0,)))

            xxcs = [xx_v[pl.ds(slot * _ITEMS + cchunk * 16, 16)]
                    for cchunk in range(16)]
            for cchunk in range(16):
                for j in range(4):
                    plsc.addupdate(d_v.at[pl.ds(rs[j] * 257 + cchunk * 16, 16)],
                                   adds[j] * xxcs[cchunk])
            for jc in range(8):
                xc = xcs[jc]
                gss = [g_v[pl.ds(rs[j] * _D + jc * 16, 16)]
                       for j in range(4)]
                ngs = [gss[j] + crs[j] * (xc - gss[j]) for j in range(4)]
                for j in range(4):
                    g_v[pl.ds(rs[j] * _D + jc * 16, 16)] = ngs[j]
            return carry

        lax.fori_loop(0, _STEPS, step, 0)
        pltpu.make_async_copy(xx_sp.at[lidx, 0],
                              xx_v.at[pl.ds(0, _ITEMS)], sem_xx).wait()
        pltpu.make_async_copy(x_sp.at[lidx, 0],
                              x_v.at[pl.ds(0, _D)], sem_x).wait()

        def acc_row(r, accs):
            return tuple(accs[j] + g_v[pl.ds(r * _D + j * 16, 16)]
                         for j in range(8))
        accs = tuple(jnp.zeros((16,), jnp.float32) for _ in range(8))
        accs = lax.fori_loop(0, _N, acc_row, accs)
        for j in range(8):
            out_v[pl.ds(j * 16, 16)] = accs[j]
        pltpu.sync_copy(out_v, out_hbm.at[b])


def kernel(embeddings, nodes):
    nodes_flat = nodes.reshape(_N, _D)
    d0, xx, sqn0 = _gram(embeddings, nodes_flat)
    out = _som_sc(
        embeddings,
        nodes_flat.reshape(_N * _D),
        sqn0.reshape(_N),
        d0.reshape(_B, _N * (_ITEMS + 1)),
        xx,
    )
    return out
